# baseline (speedup 1.0000x reference)
import functools

import jax
import jax.numpy as jnp
from jax import lax
from jax.experimental import pallas as pl
from jax.experimental.pallas import tpu as pltpu
from jax.experimental.pallas import tpu_sc as plsc

N = 4_000_000
K = 100
NC = 2
NS = 16
NW = NC * NS
PER_W = N // NW
FULL_CH = 16384
N_FULL = PER_W // FULL_CH
TAIL_CH = PER_W - N_FULL * FULL_CH
UNROLL = 8
BLK = UNROLL * 16
TAIL_BLKS = TAIL_CH // BLK
TAIL_VECS = (TAIL_CH - TAIL_BLKS * BLK) // 16
TAIL_REM = TAIL_CH - TAIL_BLKS * BLK - TAIL_VECS * 16
HBITS = 14
HBINS = 1 << HBITS
LOW_BITS = 32 - HBITS
SLICE = HBINS // NS
CAP = 256
GRP_VECS = 32
GRP = GRP_VECS * 16

_mesh = plsc.VectorSubcoreMesh(core_axis_name="c", subcore_axis_name="s")
_params = pltpu.CompilerParams(needs_layout_passes=False)


def _wid():
    return lax.axis_index("s") * NC + lax.axis_index("c")


def _key16(p):
    b = lax.bitcast_convert_type(p, jnp.uint32)
    top = b >> jnp.uint32(31)
    flip = (jnp.uint32(0) - top) | jnp.uint32(0x80000000)
    return b ^ flip


def _memset_i32(ref, nvecs, value=0):
    zz = jnp.full((16,), value, dtype=jnp.int32)

    def body(i, carry):
        ref[pl.ds(i * 16, 16)] = zz
        return carry

    lax.fori_loop(0, nvecs, body, 0)


def _chunk_loop(preds_hbm, target_hbm, base, pbufs, tbufs, sems, per_chunk,
                with_target=True):
    def issue(c):
        ch = FULL_CH if c < N_FULL else TAIL_CH
        off = base + c * FULL_CH
        b = c % 2
        dp = pltpu.async_copy(
            preds_hbm.at[pl.ds(off, ch)], pbufs[b].at[pl.ds(0, ch)], sems[b])
        if not with_target:
            return (dp,)
        dt = pltpu.async_copy(
            target_hbm.at[pl.ds(off, ch)], tbufs[b].at[pl.ds(0, ch)], sems[b])
        return dp, dt

    descs = [None, None]
    descs[0] = issue(0)
    for c in range(N_FULL + 1):
        if c + 1 <= N_FULL:
            descs[(c + 1) % 2] = issue(c + 1)
        for d in descs[c % 2]:
            d.wait()
        nblk = (FULL_CH // BLK) if c < N_FULL else TAIL_BLKS
        per_chunk(c, c % 2, nblk, is_tail=(c == N_FULL))


@functools.partial(
    pl.kernel,
    out_type=(
        jax.ShapeDtypeStruct((NW, CAP), jnp.uint32),
        jax.ShapeDtypeStruct((NW, CAP), jnp.uint32),
        jax.ShapeDtypeStruct((NW, 16), jnp.int32),
    ),
    mesh=_mesh,
    compiler_params=_params,
    scratch_types=[
        pltpu.VMEM((FULL_CH,), jnp.float32),
        pltpu.VMEM((FULL_CH,), jnp.float32),
        pltpu.VMEM((FULL_CH,), jnp.int32),
        pltpu.VMEM((FULL_CH,), jnp.int32),
        pltpu.VMEM((HBINS,), jnp.int32),
        pltpu.VMEM((NS, SLICE), jnp.int32),
        pltpu.VMEM((CAP,), jnp.uint32),
        pltpu.VMEM((CAP,), jnp.uint32),
        pltpu.VMEM((16,), jnp.int32),
        pltpu.SMEM((8,), jnp.int32),
        pltpu.VMEM_SHARED((NS, HBINS), jnp.int32),
        pltpu.VMEM_SHARED((HBINS,), jnp.int32),
        pltpu.SemaphoreType.DMA,
        pltpu.SemaphoreType.DMA,
    ],
)
def _scan(preds_hbm, target_hbm, negk_out, posk_out, cnt_out,
          pbuf0, pbuf1, tbuf0, tbuf1, hist, colbuf, negbuf, posbuf, stage,
          offs, slab, merged_sh, sem0, sem1):
    w = _wid()
    sid = lax.axis_index("s")
    base = w * PER_W
    pbufs, tbufs = (pbuf0, pbuf1), (tbuf0, tbuf1)
    lanes = lax.iota(jnp.int32, 16)

    _memset_i32(hist, HBINS // 16)

    def calc_vec(b, e16, valid=None):
        p = pbufs[b][pl.ds(e16, 16)]
        t = tbufs[b][pl.ds(e16, 16)]
        key = _key16(p)
        bucket = (key >> jnp.uint32(LOW_BITS)).astype(jnp.int32)
        negm = t == 0
        if valid is not None:
            negm = negm & valid
        cnt, last = plsc.scan_count(bucket, mask=negm)
        return bucket, cnt, last

    def do_vec(b, e16, valid=None):
        bucket, cnt, last = calc_vec(b, e16, valid)
        plsc.addupdate_scatter(hist, [bucket], cnt, mask=last)

    def per_chunk1(c, b, nblk, is_tail):
        @plsc.parallel_loop(0, nblk)
        def blk(i):
            pend = [calc_vec(b, i * BLK + u * 16) for u in range(UNROLL)]
            for bucket, cnt, last in pend:
                plsc.addupdate_scatter(hist, [bucket], cnt, mask=last)
        if is_tail:
            for u in range(TAIL_VECS):
                do_vec(b, TAIL_BLKS * BLK + u * 16)
            do_vec(b, TAIL_BLKS * BLK + TAIL_VECS * 16,
                   valid=lanes < TAIL_REM)

    _chunk_loop(preds_hbm, target_hbm, base, pbufs, tbufs,
                (sem0, sem1), per_chunk1)

    pltpu.sync_copy(hist, slab.at[sid])
    plsc.subcore_barrier()
    for r in range(NS):
        pltpu.sync_copy(slab.at[r, pl.ds(sid * SLICE, SLICE)], colbuf.at[r])

    def red(v, carry):
        acc = jnp.zeros((16,), dtype=jnp.int32)
        for r in range(NS):
            acc = acc + colbuf[r, pl.ds(v * 16, 16)]
        hist[pl.ds(sid * SLICE + v * 16, 16)] = acc
        return carry

    lax.fori_loop(0, SLICE // 16, red, 0)
    pltpu.sync_copy(hist.at[pl.ds(sid * SLICE, SLICE)],
                    merged_sh.at[pl.ds(sid * SLICE, SLICE)])
    plsc.subcore_barrier()
    pltpu.sync_copy(merged_sh, hist)

    def load_acc(vv):
        return hist[pl.ds(vv * 16, 16)]

    def walk_cond(carry):
        v, cnt_above = carry
        return (cnt_above < K) & (v < HBINS // 16)

    def walk_body(carry):
        v, cnt_above = carry
        acc = load_acc(HBINS // 16 - 1 - v)
        return v + 1, cnt_above + lax.reduce_sum(acc, axes=(0,))

    nv, cnt_incl = lax.while_loop(walk_cond, walk_body,
                                  (jnp.int32(0), jnp.int32(0)))
    vv_last = HBINS // 16 - nv
    acc = load_acc(vv_last)
    tot_last = lax.reduce_sum(acc, axes=(0,))
    suf = lax.rev(lax.cumsum(lax.rev(acc, (0,)), axis=0), (0,))
    cnt_ge = suf + jnp.full((16,), cnt_incl - tot_last, dtype=jnp.int32)
    m = lax.reduce_sum(jnp.where(cnt_ge >= K, 1, 0), axes=(0,))
    b1 = vv_last * 16 + m - 1

    def total_body(v, accv):
        return accv + load_acc(v)

    total_neg = lax.reduce_sum(
        lax.fori_loop(0, HBINS // 16, total_body,
                      jnp.zeros((16,), jnp.int32)), axes=(0,))
    lo1 = b1.astype(jnp.uint32) << jnp.uint32(LOW_BITS)
    lo1v = jnp.full((16,), lo1, dtype=jnp.uint32)

    _memset_i32(negbuf, CAP // 16)
    _memset_i32(posbuf, CAP // 16)
    offs[0] = jnp.int32(0)
    offs[1] = jnp.int32(0)

    def collect_vec(tref, t16, b, e16, key, valid=None):
        t = tref[pl.ds(t16, 16)]
        ge = key >= lo1v
        negm = (t == 0) & ge
        posm = (t == 1) & ge
        if valid is not None:
            negm = negm & valid
            posm = posm & valid
        noff = offs[0]
        poff = offs[1]
        plsc.store_compressed(negbuf.at[pl.ds(noff, 16)], key, mask=negm)
        plsc.store_compressed(posbuf.at[pl.ds(poff, 16)], key, mask=posm)
        nadd = lax.reduce_sum(jnp.where(negm, 1, 0), axes=(0,))
        padd = lax.reduce_sum(jnp.where(posm, 1, 0), axes=(0,))
        offs[0] = jnp.minimum(noff + nadd, CAP - 16)
        offs[1] = jnp.minimum(poff + padd, CAP - 16)

    def per_chunk2(c, b, nblk, is_tail):
        ngrp = nblk // (GRP_VECS // UNROLL)
        chunk_off = base + c * FULL_CH

        def grp(i, carry):
            mx = None
            for u in range(GRP_VECS):
                p = pbufs[b][pl.ds(i * GRP + u * 16, 16)]
                key = _key16(p)
                mx = key if mx is None else jnp.maximum(mx, key)
            anyhit = lax.reduce_max(mx, axes=(0,)) >= lo1

            @pl.when(anyhit)
            def _():
                pltpu.sync_copy(
                    target_hbm.at[pl.ds(chunk_off + i * GRP, GRP)],
                    tbuf0.at[pl.ds(0, GRP)])
                for u in range(GRP_VECS):
                    e16 = i * GRP + u * 16
                    collect_vec(tbuf0, u * 16, b, e16,
                                _key16(pbufs[b][pl.ds(e16, 16)]))

            return carry

        lax.fori_loop(0, ngrp, grp, 0)
        if is_tail:
            tail_off = TAIL_BLKS * BLK
            pltpu.sync_copy(
                target_hbm.at[pl.ds(chunk_off + tail_off, TAIL_VECS * 16 + TAIL_REM)],
                tbuf0.at[pl.ds(0, TAIL_VECS * 16 + TAIL_REM)])
            for u in range(TAIL_VECS):
                e16 = tail_off + u * 16
                collect_vec(tbuf0, u * 16, b, e16,
                            _key16(pbufs[b][pl.ds(e16, 16)]))
            e16 = tail_off + TAIL_VECS * 16
            collect_vec(tbuf0, TAIL_VECS * 16, b, e16,
                        _key16(pbufs[b][pl.ds(e16, 16)]),
                        valid=lanes < TAIL_REM)

    _chunk_loop(preds_hbm, target_hbm, base, pbufs, tbufs,
                (sem0, sem1), per_chunk2, with_target=False)

    lo1_i32 = lax.bitcast_convert_type(lo1v, jnp.int32)
    meta = jnp.where(lanes == 0, offs[0], 0)
    meta = meta + jnp.where(lanes == 1, offs[1], 0)
    meta = meta + jnp.where(lanes == 2, lo1_i32, 0)
    meta = meta + jnp.where(lanes == 3, total_neg, 0)
    stage[pl.ds(0, 16)] = meta
    pltpu.sync_copy(negbuf, negk_out.at[w])
    pltpu.sync_copy(posbuf, posk_out.at[w])
    pltpu.sync_copy(stage, cnt_out.at[w])


def _select_body(negk_ref, posk_ref, cnt_ref, out_ref):
    negk = negk_ref[...]
    posk = posk_ref[...]
    cnt = cnt_ref[...]
    lo1 = jnp.maximum(cnt[0, 2], cnt[1, 2]).astype(jnp.uint32)
    num_pos = N - (cnt[0, 3] + cnt[1, 3])

    def count_gt(buf, thresh):
        return jnp.sum((buf > thresh).astype(jnp.int32))

    def bis(_, carry):
        lo, hi = carry
        mid = lo + ((hi - lo) >> jnp.uint32(1))
        c = count_gt(negk, mid)
        take_hi = c < K
        lo = jnp.where(take_hi, lo, mid + jnp.uint32(1))
        hi = jnp.where(take_hi, mid, hi)
        return lo, hi

    kth, _ = lax.fori_loop(0, 32, bis, (lo1, jnp.uint32(0xFFFFFFFF)))
    hits = count_gt(posk, kth)
    res = hits.astype(jnp.float32) / num_pos.astype(jnp.float32)
    out_ref[...] = jnp.full((1, 1), res, dtype=jnp.float32)


_select = pl.pallas_call(
    _select_body,
    out_shape=jax.ShapeDtypeStruct((1, 1), jnp.float32),
)


def kernel(preds, target):
    negk, posk, cnts = _scan(preds, target)
    out = _select(negk, posk, cnts)
    return out[0, 0]


# speedup vs baseline: 1.1070x; 1.1070x over previous
import functools

import jax
import jax.numpy as jnp
from jax import lax
from jax.experimental import pallas as pl
from jax.experimental.pallas import tpu as pltpu
from jax.experimental.pallas import tpu_sc as plsc

N = 4_000_000
K = 100
NC = 2
NS = 16
NW = NC * NS
PER_W = N // NW
FULL_CH = 16384
N_FULL = PER_W // FULL_CH
TAIL_CH = PER_W - N_FULL * FULL_CH
UNROLL = 8
BLK = UNROLL * 16
TAIL_BLKS = TAIL_CH // BLK
TAIL_VECS = (TAIL_CH - TAIL_BLKS * BLK) // 16
TAIL_REM = TAIL_CH - TAIL_BLKS * BLK - TAIL_VECS * 16
HBITS = 14
HBINS = 1 << HBITS
LOW_BITS = 32 - HBITS
SLICE = HBINS // NS
CAP = 256
GRP_VECS = 32
GRP = GRP_VECS * 16

_mesh = plsc.VectorSubcoreMesh(core_axis_name="c", subcore_axis_name="s")
_params = pltpu.CompilerParams(needs_layout_passes=False)


def _wid():
    return lax.axis_index("s") * NC + lax.axis_index("c")


def _key16(p):
    b = lax.bitcast_convert_type(p, jnp.uint32)
    top = b >> jnp.uint32(31)
    flip = (jnp.uint32(0) - top) | jnp.uint32(0x80000000)
    return b ^ flip


def _memset_i32(ref, nvecs, value=0):
    zz = jnp.full((16,), value, dtype=jnp.int32)

    def body(i, carry):
        ref[pl.ds(i * 16, 16)] = zz
        return carry

    lax.fori_loop(0, nvecs, body, 0)


def _chunk_loop(preds_hbm, target_hbm, base, pbufs, tbufs, sems, per_chunk):
    def issue(c):
        ch = FULL_CH if c < N_FULL else TAIL_CH
        off = base + c * FULL_CH
        b = c % 2
        dp = pltpu.async_copy(
            preds_hbm.at[pl.ds(off, ch)], pbufs[b].at[pl.ds(0, ch)], sems[b])
        dt = pltpu.async_copy(
            target_hbm.at[pl.ds(off, ch)], tbufs[b].at[pl.ds(0, ch)], sems[b])
        return dp, dt

    descs = [None, None]
    descs[0] = issue(0)
    for c in range(N_FULL + 1):
        if c + 1 <= N_FULL:
            descs[(c + 1) % 2] = issue(c + 1)
        dp, dt = descs[c % 2]
        dp.wait()
        dt.wait()
        nblk = (FULL_CH // BLK) if c < N_FULL else TAIL_BLKS
        per_chunk(c % 2, nblk, is_tail=(c == N_FULL))


@functools.partial(
    pl.kernel,
    out_type=(
        jax.ShapeDtypeStruct((NW, CAP), jnp.uint32),
        jax.ShapeDtypeStruct((NW, CAP), jnp.uint32),
        jax.ShapeDtypeStruct((NW, 16), jnp.int32),
    ),
    mesh=_mesh,
    compiler_params=_params,
    scratch_types=[
        pltpu.VMEM((FULL_CH,), jnp.float32),
        pltpu.VMEM((FULL_CH,), jnp.float32),
        pltpu.VMEM((FULL_CH,), jnp.int32),
        pltpu.VMEM((FULL_CH,), jnp.int32),
        pltpu.VMEM((HBINS,), jnp.int32),
        pltpu.VMEM((NS, SLICE), jnp.int32),
        pltpu.VMEM((CAP,), jnp.uint32),
        pltpu.VMEM((CAP,), jnp.uint32),
        pltpu.VMEM((16,), jnp.int32),
        pltpu.SMEM((8,), jnp.int32),
        pltpu.VMEM_SHARED((NS, HBINS), jnp.int32),
        pltpu.VMEM_SHARED((HBINS,), jnp.int32),
        pltpu.SemaphoreType.DMA,
        pltpu.SemaphoreType.DMA,
    ],
)
def _scan(preds_hbm, target_hbm, negk_out, posk_out, cnt_out,
          pbuf0, pbuf1, tbuf0, tbuf1, hist, colbuf, negbuf, posbuf, stage,
          offs, slab, merged_sh, sem0, sem1):
    w = _wid()
    sid = lax.axis_index("s")
    base = w * PER_W
    pbufs, tbufs = (pbuf0, pbuf1), (tbuf0, tbuf1)
    lanes = lax.iota(jnp.int32, 16)

    _memset_i32(hist, HBINS // 16)

    def calc_vec(b, e16, valid=None):
        p = pbufs[b][pl.ds(e16, 16)]
        t = tbufs[b][pl.ds(e16, 16)]
        key = _key16(p)
        bucket = (key >> jnp.uint32(LOW_BITS)).astype(jnp.int32)
        negm = t == 0
        if valid is not None:
            negm = negm & valid
        cnt, last = plsc.scan_count(bucket, mask=negm)
        return bucket, cnt, last

    def do_vec(b, e16, valid=None):
        bucket, cnt, last = calc_vec(b, e16, valid)
        plsc.addupdate_scatter(hist, [bucket], cnt, mask=last)

    def per_chunk1(b, nblk, is_tail):
        @plsc.parallel_loop(0, nblk)
        def blk(i):
            pend = [calc_vec(b, i * BLK + u * 16) for u in range(UNROLL)]
            for bucket, cnt, last in pend:
                plsc.addupdate_scatter(hist, [bucket], cnt, mask=last)
        if is_tail:
            for u in range(TAIL_VECS):
                do_vec(b, TAIL_BLKS * BLK + u * 16)
            do_vec(b, TAIL_BLKS * BLK + TAIL_VECS * 16,
                   valid=lanes < TAIL_REM)

    _chunk_loop(preds_hbm, target_hbm, base, pbufs, tbufs,
                (sem0, sem1), per_chunk1)

    pltpu.sync_copy(hist, slab.at[sid])
    plsc.subcore_barrier()
    for r in range(NS):
        pltpu.sync_copy(slab.at[r, pl.ds(sid * SLICE, SLICE)], colbuf.at[r])

    def red(v, carry):
        acc = jnp.zeros((16,), dtype=jnp.int32)
        for r in range(NS):
            acc = acc + colbuf[r, pl.ds(v * 16, 16)]
        hist[pl.ds(sid * SLICE + v * 16, 16)] = acc
        return carry

    lax.fori_loop(0, SLICE // 16, red, 0)
    pltpu.sync_copy(hist.at[pl.ds(sid * SLICE, SLICE)],
                    merged_sh.at[pl.ds(sid * SLICE, SLICE)])
    plsc.subcore_barrier()
    pltpu.sync_copy(merged_sh, hist)

    def load_acc(vv):
        return hist[pl.ds(vv * 16, 16)]

    def walk_cond(carry):
        v, cnt_above = carry
        return (cnt_above < K) & (v < HBINS // 16)

    def walk_body(carry):
        v, cnt_above = carry
        acc = load_acc(HBINS // 16 - 1 - v)
        return v + 1, cnt_above + lax.reduce_sum(acc, axes=(0,))

    nv, cnt_incl = lax.while_loop(walk_cond, walk_body,
                                  (jnp.int32(0), jnp.int32(0)))
    vv_last = HBINS // 16 - nv
    acc = load_acc(vv_last)
    tot_last = lax.reduce_sum(acc, axes=(0,))
    suf = lax.rev(lax.cumsum(lax.rev(acc, (0,)), axis=0), (0,))
    cnt_ge = suf + jnp.full((16,), cnt_incl - tot_last, dtype=jnp.int32)
    m = lax.reduce_sum(jnp.where(cnt_ge >= K, 1, 0), axes=(0,))
    b1 = vv_last * 16 + m - 1

    def total_body(v, accv):
        return accv + load_acc(v)

    total_neg = lax.reduce_sum(
        lax.fori_loop(0, HBINS // 16, total_body,
                      jnp.zeros((16,), jnp.int32)), axes=(0,))
    lo1 = b1.astype(jnp.uint32) << jnp.uint32(LOW_BITS)
    lo1v = jnp.full((16,), lo1, dtype=jnp.uint32)

    _memset_i32(negbuf, CAP // 16)
    _memset_i32(posbuf, CAP // 16)
    offs[0] = jnp.int32(0)
    offs[1] = jnp.int32(0)

    def collect_vec(b, e16, key, valid=None):
        t = tbufs[b][pl.ds(e16, 16)]
        ge = key >= lo1v
        negm = (t == 0) & ge
        posm = (t == 1) & ge
        if valid is not None:
            negm = negm & valid
            posm = posm & valid
        noff = offs[0]
        poff = offs[1]
        plsc.store_compressed(negbuf.at[pl.ds(noff, 16)], key, mask=negm)
        plsc.store_compressed(posbuf.at[pl.ds(poff, 16)], key, mask=posm)
        nadd = lax.reduce_sum(jnp.where(negm, 1, 0), axes=(0,))
        padd = lax.reduce_sum(jnp.where(posm, 1, 0), axes=(0,))
        offs[0] = jnp.minimum(noff + nadd, CAP - 16)
        offs[1] = jnp.minimum(poff + padd, CAP - 16)

    def per_chunk2(b, nblk, is_tail):
        ngrp = nblk // (GRP_VECS // UNROLL)

        def grp(i, carry):
            mx = None
            for u in range(GRP_VECS):
                p = pbufs[b][pl.ds(i * GRP + u * 16, 16)]
                key = _key16(p)
                mx = key if mx is None else jnp.maximum(mx, key)
            anyhit = lax.reduce_max(mx, axes=(0,)) >= lo1

            @pl.when(anyhit)
            def _():
                for u in range(GRP_VECS):
                    e16 = i * GRP + u * 16
                    collect_vec(b, e16, _key16(pbufs[b][pl.ds(e16, 16)]))

            return carry

        lax.fori_loop(0, ngrp, grp, 0)
        if is_tail:
            for u in range(TAIL_VECS):
                e16 = TAIL_BLKS * BLK + u * 16
                collect_vec(b, e16, _key16(pbufs[b][pl.ds(e16, 16)]))
            e16 = TAIL_BLKS * BLK + TAIL_VECS * 16
            collect_vec(b, e16, _key16(pbufs[b][pl.ds(e16, 16)]),
                        valid=lanes < TAIL_REM)

    _chunk_loop(preds_hbm, target_hbm, base, pbufs, tbufs,
                (sem0, sem1), per_chunk2)

    lo1_i32 = lax.bitcast_convert_type(lo1v, jnp.int32)
    meta = jnp.where(lanes == 0, offs[0], 0)
    meta = meta + jnp.where(lanes == 1, offs[1], 0)
    meta = meta + jnp.where(lanes == 2, lo1_i32, 0)
    meta = meta + jnp.where(lanes == 3, total_neg, 0)
    stage[pl.ds(0, 16)] = meta
    pltpu.sync_copy(negbuf, negk_out.at[w])
    pltpu.sync_copy(posbuf, posk_out.at[w])
    pltpu.sync_copy(stage, cnt_out.at[w])


def _select_body(negk_ref, posk_ref, cnt_ref, out_ref):
    negk = negk_ref[...]
    posk = posk_ref[...]
    cnt = cnt_ref[...]
    lo1 = jnp.maximum(cnt[0, 2], cnt[1, 2]).astype(jnp.uint32)
    num_pos = N - (cnt[0, 3] + cnt[1, 3])

    def count_gt(buf, thresh):
        return jnp.sum((buf > thresh).astype(jnp.int32))

    def bis(_, carry):
        lo, hi = carry
        mid = lo + ((hi - lo) >> jnp.uint32(1))
        c = count_gt(negk, mid)
        take_hi = c < K
        lo = jnp.where(take_hi, lo, mid + jnp.uint32(1))
        hi = jnp.where(take_hi, mid, hi)
        return lo, hi

    kth, _ = lax.fori_loop(0, 32, bis, (lo1, jnp.uint32(0xFFFFFFFF)))
    hits = count_gt(posk, kth)
    res = hits.astype(jnp.float32) / num_pos.astype(jnp.float32)
    out_ref[...] = jnp.full((1, 1), res, dtype=jnp.float32)


_select = pl.pallas_call(
    _select_body,
    out_shape=jax.ShapeDtypeStruct((1, 1), jnp.float32),
)


def kernel(preds, target):
    negk, posk, cnts = _scan(preds, target)
    out = _select(negk, posk, cnts)
    return out[0, 0]


# speedup vs baseline: 1.1151x; 1.0073x over previous
import functools

import jax
import jax.numpy as jnp
from jax import lax
from jax.experimental import pallas as pl
from jax.experimental.pallas import tpu as pltpu
from jax.experimental.pallas import tpu_sc as plsc

N = 4_000_000
K = 100
NC = 2
NS = 16
NW = NC * NS
PER_W = N // NW
FULL_CH = 16384
N_FULL = PER_W // FULL_CH
TAIL_CH = PER_W - N_FULL * FULL_CH
UNROLL = 8
BLK = UNROLL * 16
TAIL_BLKS = TAIL_CH // BLK
TAIL_VECS = (TAIL_CH - TAIL_BLKS * BLK) // 16
TAIL_REM = TAIL_CH - TAIL_BLKS * BLK - TAIL_VECS * 16
HBITS = 14
HBINS = 1 << HBITS
LOW_BITS = 32 - HBITS
SLICE = HBINS // NS
CAP = 256
GRP_VECS = 32
GRP = GRP_VECS * 16

_mesh = plsc.VectorSubcoreMesh(core_axis_name="c", subcore_axis_name="s")
_params = pltpu.CompilerParams(needs_layout_passes=False)


def _wid():
    return lax.axis_index("s") * NC + lax.axis_index("c")


def _key16(p):
    b = lax.bitcast_convert_type(p, jnp.uint32)
    top = b >> jnp.uint32(31)
    flip = (jnp.uint32(0) - top) | jnp.uint32(0x80000000)
    return b ^ flip


def _memset_i32(ref, nvecs, value=0):
    zz = jnp.full((16,), value, dtype=jnp.int32)

    def body(i, carry):
        ref[pl.ds(i * 16, 16)] = zz
        return carry

    lax.fori_loop(0, nvecs, body, 0)


def _chunk_loop(preds_hbm, target_hbm, base, pbufs, tbufs, sems, per_chunk):
    def issue(c):
        ch = FULL_CH if c < N_FULL else TAIL_CH
        off = base + c * FULL_CH
        b = c % 2
        dp = pltpu.async_copy(
            preds_hbm.at[pl.ds(off, ch)], pbufs[b].at[pl.ds(0, ch)], sems[b])
        dt = pltpu.async_copy(
            target_hbm.at[pl.ds(off, ch)], tbufs[b].at[pl.ds(0, ch)], sems[b])
        return dp, dt

    descs = [None, None]
    descs[0] = issue(0)
    for c in range(N_FULL + 1):
        if c + 1 <= N_FULL:
            descs[(c + 1) % 2] = issue(c + 1)
        dp, dt = descs[c % 2]
        dp.wait()
        dt.wait()
        nblk = (FULL_CH // BLK) if c < N_FULL else TAIL_BLKS
        per_chunk(c % 2, nblk, is_tail=(c == N_FULL))


@functools.partial(
    pl.kernel,
    out_type=(
        jax.ShapeDtypeStruct((NW, CAP), jnp.uint32),
        jax.ShapeDtypeStruct((NW, CAP), jnp.uint32),
        jax.ShapeDtypeStruct((NW, 16), jnp.int32),
    ),
    mesh=_mesh,
    compiler_params=_params,
    scratch_types=[
        pltpu.VMEM((FULL_CH,), jnp.float32),
        pltpu.VMEM((FULL_CH,), jnp.float32),
        pltpu.VMEM((FULL_CH,), jnp.int32),
        pltpu.VMEM((FULL_CH,), jnp.int32),
        pltpu.VMEM((HBINS,), jnp.int32),
        pltpu.VMEM((NS, SLICE), jnp.int32),
        pltpu.VMEM((CAP,), jnp.uint32),
        pltpu.VMEM((CAP,), jnp.uint32),
        pltpu.VMEM((16,), jnp.int32),
        pltpu.SMEM((8,), jnp.int32),
        pltpu.VMEM_SHARED((NS, HBINS), jnp.int32),
        pltpu.VMEM_SHARED((HBINS,), jnp.int32),
        pltpu.SemaphoreType.DMA,
        pltpu.SemaphoreType.DMA,
    ],
)
def _scan(preds_hbm, target_hbm, negk_out, posk_out, cnt_out,
          pbuf0, pbuf1, tbuf0, tbuf1, hist, colbuf, negbuf, posbuf, stage,
          offs, slab, merged_sh, sem0, sem1):
    w = _wid()
    sid = lax.axis_index("s")
    base = w * PER_W
    pbufs, tbufs = (pbuf0, pbuf1), (tbuf0, tbuf1)
    lanes = lax.iota(jnp.int32, 16)

    _memset_i32(hist, HBINS // 16)

    def calc_vec(b, e16, valid=None):
        p = pbufs[b][pl.ds(e16, 16)]
        t = tbufs[b][pl.ds(e16, 16)]
        key = _key16(p)
        bucket = (key >> jnp.uint32(LOW_BITS)).astype(jnp.int32)
        negm = t == 0
        if valid is not None:
            negm = negm & valid
        cnt, last = plsc.scan_count(bucket, mask=negm)
        return bucket, cnt, last

    def do_vec(b, e16, valid=None):
        bucket, cnt, last = calc_vec(b, e16, valid)
        plsc.addupdate_scatter(hist, [bucket], cnt, mask=last)

    def per_chunk1(b, nblk, is_tail):
        @plsc.parallel_loop(0, nblk)
        def blk(i):
            pend = [calc_vec(b, i * BLK + u * 16) for u in range(UNROLL)]
            for bucket, cnt, last in pend:
                plsc.addupdate_scatter(hist, [bucket], cnt, mask=last)
        if is_tail:
            for u in range(TAIL_VECS):
                do_vec(b, TAIL_BLKS * BLK + u * 16)
            do_vec(b, TAIL_BLKS * BLK + TAIL_VECS * 16,
                   valid=lanes < TAIL_REM)

    _chunk_loop(preds_hbm, target_hbm, base, pbufs, tbufs,
                (sem0, sem1), per_chunk1)

    pltpu.sync_copy(hist, slab.at[sid])
    plsc.subcore_barrier()
    for r in range(NS):
        pltpu.sync_copy(slab.at[r, pl.ds(sid * SLICE, SLICE)], colbuf.at[r])

    def red(v, carry):
        acc = jnp.zeros((16,), dtype=jnp.int32)
        for r in range(NS):
            acc = acc + colbuf[r, pl.ds(v * 16, 16)]
        hist[pl.ds(sid * SLICE + v * 16, 16)] = acc
        return carry

    lax.fori_loop(0, SLICE // 16, red, 0)
    pltpu.sync_copy(hist.at[pl.ds(sid * SLICE, SLICE)],
                    merged_sh.at[pl.ds(sid * SLICE, SLICE)])
    plsc.subcore_barrier()
    pltpu.sync_copy(merged_sh, hist)

    def load_acc(vv):
        return hist[pl.ds(vv * 16, 16)]

    def walk_cond(carry):
        v, cnt_above = carry
        return (cnt_above < K) & (v < HBINS // 16)

    def walk_body(carry):
        v, cnt_above = carry
        acc = load_acc(HBINS // 16 - 1 - v)
        return v + 1, cnt_above + lax.reduce_sum(acc, axes=(0,))

    nv, cnt_incl = lax.while_loop(walk_cond, walk_body,
                                  (jnp.int32(0), jnp.int32(0)))
    vv_last = HBINS // 16 - nv
    acc = load_acc(vv_last)
    tot_last = lax.reduce_sum(acc, axes=(0,))
    suf = lax.rev(lax.cumsum(lax.rev(acc, (0,)), axis=0), (0,))
    cnt_ge = suf + jnp.full((16,), cnt_incl - tot_last, dtype=jnp.int32)
    m = lax.reduce_sum(jnp.where(cnt_ge >= K, 1, 0), axes=(0,))
    b1 = vv_last * 16 + m - 1

    def total_body(v, accv):
        return accv + load_acc(v)

    total_neg = lax.reduce_sum(
        lax.fori_loop(0, HBINS // 16, total_body,
                      jnp.zeros((16,), jnp.int32)), axes=(0,))
    lo1 = b1.astype(jnp.uint32) << jnp.uint32(LOW_BITS)
    lo1v = jnp.full((16,), lo1, dtype=jnp.uint32)

    _memset_i32(negbuf, CAP // 16)
    _memset_i32(posbuf, CAP // 16)
    offs[0] = jnp.int32(0)
    offs[1] = jnp.int32(0)

    def collect_vec(b, e16, key, valid=None):
        t = tbufs[b][pl.ds(e16, 16)]
        ge = key >= lo1v
        negm = (t == 0) & ge
        posm = (t == 1) & ge
        if valid is not None:
            negm = negm & valid
            posm = posm & valid
        noff = offs[0]
        poff = offs[1]
        plsc.store_compressed(negbuf.at[pl.ds(noff, 16)], key, mask=negm)
        plsc.store_compressed(posbuf.at[pl.ds(poff, 16)], key, mask=posm)
        nadd = lax.reduce_sum(jnp.where(negm, 1, 0), axes=(0,))
        padd = lax.reduce_sum(jnp.where(posm, 1, 0), axes=(0,))
        offs[0] = jnp.minimum(noff + nadd, CAP - 16)
        offs[1] = jnp.minimum(poff + padd, CAP - 16)

    def per_chunk2(b, nblk, is_tail):
        ngrp = nblk // (GRP_VECS // UNROLL)

        def grp(i, carry):
            mx = None
            for u in range(GRP_VECS):
                p = pbufs[b][pl.ds(i * GRP + u * 16, 16)]
                key = _key16(p)
                mx = key if mx is None else jnp.maximum(mx, key)
            pc = plsc.all_reduce_population_count(mx >= lo1v)
            anyhit = pc[0] > 0

            @pl.when(anyhit)
            def _():
                for u in range(GRP_VECS):
                    e16 = i * GRP + u * 16
                    collect_vec(b, e16, _key16(pbufs[b][pl.ds(e16, 16)]))

            return carry

        lax.fori_loop(0, ngrp, grp, 0)
        if is_tail:
            for u in range(TAIL_VECS):
                e16 = TAIL_BLKS * BLK + u * 16
                collect_vec(b, e16, _key16(pbufs[b][pl.ds(e16, 16)]))
            e16 = TAIL_BLKS * BLK + TAIL_VECS * 16
            collect_vec(b, e16, _key16(pbufs[b][pl.ds(e16, 16)]),
                        valid=lanes < TAIL_REM)

    _chunk_loop(preds_hbm, target_hbm, base, pbufs, tbufs,
                (sem0, sem1), per_chunk2)

    lo1_i32 = lax.bitcast_convert_type(lo1v, jnp.int32)
    meta = jnp.where(lanes == 0, offs[0], 0)
    meta = meta + jnp.where(lanes == 1, offs[1], 0)
    meta = meta + jnp.where(lanes == 2, lo1_i32, 0)
    meta = meta + jnp.where(lanes == 3, total_neg, 0)
    stage[pl.ds(0, 16)] = meta
    pltpu.sync_copy(negbuf, negk_out.at[w])
    pltpu.sync_copy(posbuf, posk_out.at[w])
    pltpu.sync_copy(stage, cnt_out.at[w])


def _select_body(negk_ref, posk_ref, cnt_ref, out_ref):
    negk = negk_ref[...]
    posk = posk_ref[...]
    cnt = cnt_ref[...]
    lo1 = jnp.maximum(cnt[0, 2], cnt[1, 2]).astype(jnp.uint32)
    num_pos = N - (cnt[0, 3] + cnt[1, 3])

    def count_gt(buf, thresh):
        return jnp.sum((buf > thresh).astype(jnp.int32))

    def bis(_, carry):
        lo, hi = carry
        mid = lo + ((hi - lo) >> jnp.uint32(1))
        c = count_gt(negk, mid)
        take_hi = c < K
        lo = jnp.where(take_hi, lo, mid + jnp.uint32(1))
        hi = jnp.where(take_hi, mid, hi)
        return lo, hi

    kth, _ = lax.fori_loop(0, 32, bis, (lo1, jnp.uint32(0xFFFFFFFF)))
    hits = count_gt(posk, kth)
    res = hits.astype(jnp.float32) / num_pos.astype(jnp.float32)
    out_ref[...] = jnp.full((1, 1), res, dtype=jnp.float32)


_select = pl.pallas_call(
    _select_body,
    out_shape=jax.ShapeDtypeStruct((1, 1), jnp.float32),
)


def kernel(preds, target):
    negk, posk, cnts = _scan(preds, target)
    out = _select(negk, posk, cnts)
    return out[0, 0]
